# speedup vs baseline: 6.3613x; 6.3613x over previous
import functools

import jax
import jax.numpy as jnp
from jax import lax
from jax.experimental import pallas as pl
from jax.experimental.pallas import tpu as pltpu
from jax.experimental.pallas import tpu_sc as plsc

NUM_EMB = 100000
DIM = 128
BATCH = 4096
SEQ = 200
B_TOTAL = BATCH * SEQ

NC = 2
NS = 16
NW = NC * NS
ROWS_PER_W = B_TOTAL // NW
GROW = 128
K = ROWS_PER_W // GROW

_mesh = plsc.VectorSubcoreMesh(
    core_axis_name="c", subcore_axis_name="s", num_cores=NC, num_subcores=NS
)


@functools.partial(
    pl.kernel,
    out_type=jax.ShapeDtypeStruct((B_TOTAL, DIM), jnp.float32),
    mesh=_mesh,
    scratch_types=[
        pltpu.VMEM((K, GROW), jnp.int32),
        pltpu.VMEM((GROW, DIM), jnp.float32),
        pltpu.SemaphoreType.DMA,
    ],
)
def _gather_kernel(idx_hbm, table_hbm, out_hbm, idx_v, rows_v, sem):
    wid = lax.axis_index("s") * NC + lax.axis_index("c")
    base = wid * ROWS_PER_W
    pltpu.sync_copy(idx_hbm.at[wid], idx_v)

    def body(j, carry):
        pltpu.async_copy(table_hbm.at[idx_v.at[j]], rows_v, sem).wait()
        pltpu.sync_copy(rows_v, out_hbm.at[pl.ds(base + j * GROW, GROW)])
        return carry

    lax.fori_loop(0, K, body, 0, unroll=False)


def kernel(input_ids, weight):
    idx = input_ids.reshape(NW, K, GROW)
    out = _gather_kernel(idx, weight)
    return out.reshape(BATCH, SEQ, DIM)


# speedup vs baseline: 9.2651x; 1.4565x over previous
import functools

import jax
import jax.numpy as jnp
from jax import lax
from jax.experimental import pallas as pl
from jax.experimental.pallas import tpu as pltpu
from jax.experimental.pallas import tpu_sc as plsc

NUM_EMB = 100000
DIM = 128
BATCH = 4096
SEQ = 200
B_TOTAL = BATCH * SEQ

NC = 2
NS = 16
NW = NC * NS
ROWS_PER_W = B_TOTAL // NW
GROW = 128
K = ROWS_PER_W // GROW

_mesh = plsc.VectorSubcoreMesh(
    core_axis_name="c", subcore_axis_name="s", num_cores=NC, num_subcores=NS
)


@functools.partial(
    pl.kernel,
    out_type=jax.ShapeDtypeStruct((B_TOTAL, DIM), jnp.float32),
    mesh=_mesh,
    scratch_types=[
        pltpu.VMEM((K, GROW), jnp.int32),
        pltpu.VMEM((GROW, DIM), jnp.float32),
        pltpu.VMEM((GROW, DIM), jnp.float32),
        pltpu.SemaphoreType.DMA,
        pltpu.SemaphoreType.DMA,
    ],
)
def _gather_kernel(idx_hbm, table_hbm, out_hbm, idx_v, rows0, rows1, sem0, sem1):
    wid = lax.axis_index("s") * NC + lax.axis_index("c")
    base = wid * ROWS_PER_W
    pltpu.sync_copy(idx_hbm.at[wid], idx_v)

    rows = (rows0, rows1)
    sems = (sem0, sem1)

    pltpu.async_copy(table_hbm.at[idx_v.at[0]], rows0, sem0)
    pltpu.async_copy(table_hbm.at[idx_v.at[1]], rows1, sem1)

    def body(g, carry):
        for b in range(2):
            j = 2 * g + b
            pltpu.make_async_copy(table_hbm.at[idx_v.at[j]], rows[b], sems[b]).wait()
            pltpu.sync_copy(rows[b], out_hbm.at[pl.ds(base + j * GROW, GROW)])
            @pl.when(j + 2 < K)
            def _():
                pltpu.async_copy(table_hbm.at[idx_v.at[j + 2]], rows[b], sems[b])
        return carry

    lax.fori_loop(0, K // 2, body, 0, unroll=False)


def kernel(input_ids, weight):
    idx = input_ids.reshape(NW, K, GROW)
    out = _gather_kernel(idx, weight)
    return out.reshape(BATCH, SEQ, DIM)
